# baseline (speedup 1.0000x reference)
import functools

import jax
import jax.numpy as jnp
from jax import lax
from jax.experimental import pallas as pl
from jax.experimental.pallas import tpu as pltpu
from jax.experimental.pallas import tpu_sc as plsc

B, L = 4096, 200
VOCAB, HIDDEN, NUM_CLASSES = 1000000, 32, 2
HALF_L = L // 2

NUM_CORES, NUM_SUBCORES, LANES = 2, 16, 16
NUM_WORKERS = NUM_CORES * NUM_SUBCORES
SEQ_PER_W = B // NUM_WORKERS
OUT_PAD = LANES
NBUF = 4
FULL_GROUPS = L // LANES
REM = L % LANES

PROJ_BLK = 16384
PROJ_GRID = -(-VOCAB // PROJ_BLK)

_mesh = plsc.VectorSubcoreMesh(
    core_axis_name="c", subcore_axis_name="s",
    num_cores=NUM_CORES, num_subcores=NUM_SUBCORES,
)


def _project_body(wpad_ref, embt_ref, out_ref):
    y = jnp.dot(wpad_ref[...], embt_ref[...],
                preferred_element_type=jnp.float32)
    out_ref[...] = y.T


_project = pl.pallas_call(
    _project_body,
    grid=(PROJ_GRID,),
    in_specs=[
        pl.BlockSpec((LANES, HIDDEN), lambda i: (0, 0)),
        pl.BlockSpec((HIDDEN, PROJ_BLK), lambda i: (0, i)),
    ],
    out_specs=pl.BlockSpec((PROJ_BLK, LANES), lambda i: (i, 0)),
    out_shape=jax.ShapeDtypeStruct((VOCAB, LANES), jnp.float32),
)


@functools.partial(
    pl.kernel,
    out_type=jax.ShapeDtypeStruct((B, OUT_PAD), jnp.float32),
    mesh=_mesh,
    compiler_params=pltpu.CompilerParams(
        needs_layout_passes=False, use_tc_tiling_on_sc=False),
    scratch_types=[
        pltpu.VMEM((SEQ_PER_W, 2, HALF_L), jnp.int32),
        pltpu.VMEM((SEQ_PER_W, L), jnp.float32),
        pltpu.VMEM((NBUF, L, LANES), jnp.float32),
        pltpu.VMEM((SEQ_PER_W, OUT_PAD), jnp.float32),
        pltpu.VMEM((LANES,), jnp.float32),
        pltpu.SemaphoreType.DMA,
        pltpu.SemaphoreType.DMA,
        pltpu.SemaphoreType.DMA,
        pltpu.SemaphoreType.DMA,
    ],
)
def _sc_pool(ids_hbm, mask_hbm, p_hbm, b_hbm, out_hbm,
             ids_v, mask_v, rows_v, out_v, b_v,
             sem0, sem1, sem2, sem3):
    sems = (sem0, sem1, sem2, sem3)
    wid = lax.axis_index("s") * NUM_CORES + lax.axis_index("c")
    base = wid * SEQ_PER_W

    pltpu.sync_copy(ids_hbm.at[pl.ds(base, SEQ_PER_W)], ids_v)
    pltpu.sync_copy(mask_hbm.at[pl.ds(base, SEQ_PER_W)], mask_v)
    pltpu.sync_copy(b_hbm, b_v)

    b_vec = b_v[pl.ds(0, LANES)]
    zero = jnp.zeros((LANES,), jnp.float32)
    lane = lax.iota(jnp.int32, LANES)

    def copies(j, buf):
        return (
            pltpu.make_async_copy(
                p_hbm.at[ids_v.at[j, 0]],
                rows_v.at[buf, pl.ds(0, HALF_L)], sems[buf]),
            pltpu.make_async_copy(
                p_hbm.at[ids_v.at[j, 1]],
                rows_v.at[buf, pl.ds(HALF_L, HALF_L)], sems[buf]),
        )

    def fire(j, buf):
        for cp in copies(j, buf):
            cp.start()

    def drain(j, buf):
        for cp in copies(j, buf):
            cp.wait()

    def compute(j, buf):
        def group_body(g, carry):
            acc, msvec = carry
            mvec = mask_v[j, pl.ds(g * LANES, LANES)]
            t0 = g * LANES
            for i in range(LANES):
                acc = acc + rows_v[buf, t0 + i, pl.ds(0, LANES)] * mvec[i]
            return (acc, msvec + mvec)

        acc, msvec = lax.fori_loop(0, FULL_GROUPS, group_body, (zero, zero))

        mvec = mask_v[j, pl.ds(L - LANES, LANES)]
        for i in range(LANES - REM, LANES):
            t = L - LANES + i
            acc = acc + rows_v[buf, t, pl.ds(0, LANES)] * mvec[i]
        msvec = msvec + jnp.where(lane >= LANES - REM, mvec, 0.0)

        inv = jnp.full((LANES,), 1.0, jnp.float32) / jnp.broadcast_to(
            jnp.sum(msvec), (LANES,))
        out_v[j, pl.ds(0, LANES)] = acc * inv + b_vec

    for buf in range(NBUF):
        fire(jnp.int32(buf), buf)

    def ring_body(g, carry):
        j0 = g * NBUF
        for buf in range(NBUF):
            j = j0 + buf
            drain(j, buf)
            compute(j, buf)
            nxt = j + NBUF

            @pl.when(nxt < SEQ_PER_W)
            def _():
                fire(nxt, buf)
        return carry

    lax.fori_loop(0, SEQ_PER_W // NBUF, ring_body, jnp.int32(0))

    pltpu.sync_copy(out_v, out_hbm.at[pl.ds(base, SEQ_PER_W)])


def kernel(input_ids, attention_mask, embedding, W, b):
    ids = input_ids.astype(jnp.int32).reshape(B, 2, HALF_L)
    w_pad = jnp.zeros((LANES, HIDDEN), jnp.float32).at[:NUM_CLASSES].set(
        W.astype(jnp.float32))
    b_pad = jnp.zeros((LANES,), jnp.float32).at[:NUM_CLASSES].set(
        b.astype(jnp.float32))
    proj = _project(w_pad, embedding.T)
    padded = _sc_pool(ids, attention_mask.astype(jnp.float32), proj, b_pad)
    return padded[:, :NUM_CLASSES]
